# baseline (speedup 1.0000x reference)
import functools

import jax
import jax.numpy as jnp
from jax.experimental import pallas as pl

_NUM_CLASS = 1000


def _ploss_block(labels_ref, out_ref, gl_ref, acc_ref, *, nblocks, n_rows):
    i = pl.program_id(0)
    x = out_ref[...]
    gl = gl_ref[...]
    lbl = labels_ref[...]

    dot = jax.lax.dot_general(
        x, gl, (((1,), (1,)), ((), ())),
        preferred_element_type=jnp.float32,
        precision=jax.lax.Precision.HIGHEST,
    )
    a2 = jnp.sum(x * x, axis=1, keepdims=True)
    glsq = gl * gl
    ones = jnp.ones((1, glsq.shape[0]), jnp.float32)
    b2 = jax.lax.dot_general(
        ones, glsq, (((1,), (1,)), ((), ())),
        preferred_element_type=jnp.float32,
        precision=jax.lax.Precision.HIGHEST,
    )
    d2 = jnp.maximum(a2 + b2 - 2.0 * dot, 1e-12)

    m = jnp.min(d2, axis=1, keepdims=True)
    iota = jax.lax.broadcasted_iota(jnp.int32, d2.shape, 1)
    idx = jnp.min(jnp.where(d2 == m, iota, d2.shape[1]), axis=1,
                  keepdims=True)

    label_used = jnp.where(lbl > _NUM_CLASS - 1, idx, lbl)

    mx = jnp.max(x, axis=1, keepdims=True)
    lse = jnp.log(jnp.sum(jnp.exp(x - mx), axis=1, keepdims=True)) + mx

    xiota = jax.lax.broadcasted_iota(jnp.int32, x.shape, 1)
    picked = jnp.sum(jnp.where(xiota == label_used, x, 0.0), axis=1,
                     keepdims=True)

    part = jnp.sum(lse - picked, axis=0, keepdims=True)

    @pl.when(i == 0)
    def _():
        acc_ref[...] = jnp.zeros_like(acc_ref)

    acc_ref[...] += part

    @pl.when(i == nblocks - 1)
    def _():
        acc_ref[...] = acc_ref[...] * (1.0 / n_rows)


@functools.partial(jax.jit, static_argnames=())
def _ploss(outputs, labels, global_logit):
    n, k = outputs.shape
    blk = 512
    nblocks = n // blk
    labels2d = labels.reshape(n, 1)
    out = pl.pallas_call(
        functools.partial(_ploss_block, nblocks=nblocks, n_rows=n),
        grid=(nblocks,),
        in_specs=[
            pl.BlockSpec((blk, 1), lambda i: (i, 0)),
            pl.BlockSpec((blk, k), lambda i: (i, 0)),
            pl.BlockSpec(global_logit.shape, lambda i: (0, 0)),
        ],
        out_specs=pl.BlockSpec((1, 1), lambda i: (0, 0)),
        out_shape=jax.ShapeDtypeStruct((1, 1), jnp.float32),
    )(labels2d, outputs, global_logit)
    return out[0, 0]


def kernel(outputs, labels, global_logit):
    return _ploss(outputs.astype(jnp.float32), labels, global_logit)


# speedup vs baseline: 2.3220x; 2.3220x over previous
import functools

import jax
import jax.numpy as jnp
from jax.experimental import pallas as pl

_NUM_CLASS = 1000


def _ploss_block(labels_ref, out_ref, gl_ref, acc_ref, *, nblocks, n_rows):
    i = pl.program_id(0)
    x = out_ref[...]
    gl = gl_ref[...]
    lbl = labels_ref[...]

    dot = jax.lax.dot_general(
        x, gl, (((1,), (1,)), ((), ())),
        preferred_element_type=jnp.float32,
    )
    a2 = jnp.sum(x * x, axis=1, keepdims=True)
    glsq = gl * gl
    ones = jnp.ones((1, glsq.shape[0]), jnp.float32)
    b2 = jax.lax.dot_general(
        ones, glsq, (((1,), (1,)), ((), ())),
        preferred_element_type=jnp.float32,
    )
    d2 = jnp.maximum(a2 + b2 - 2.0 * dot, 1e-12)

    m = jnp.min(d2, axis=1, keepdims=True)
    iota = jax.lax.broadcasted_iota(jnp.int32, d2.shape, 1)
    idx = jnp.min(jnp.where(d2 == m, iota, d2.shape[1]), axis=1,
                  keepdims=True)

    label_used = jnp.where(lbl > _NUM_CLASS - 1, idx, lbl)

    mx = jnp.max(x, axis=1, keepdims=True)
    lse = jnp.log(jnp.sum(jnp.exp(x - mx), axis=1, keepdims=True)) + mx

    xiota = jax.lax.broadcasted_iota(jnp.int32, x.shape, 1)
    picked = jnp.sum(jnp.where(xiota == label_used, x, 0.0), axis=1,
                     keepdims=True)

    part = jnp.sum(lse - picked, axis=0, keepdims=True)

    @pl.when(i == 0)
    def _():
        acc_ref[...] = jnp.zeros_like(acc_ref)

    acc_ref[...] += part

    @pl.when(i == nblocks - 1)
    def _():
        acc_ref[...] = acc_ref[...] * (1.0 / n_rows)


@functools.partial(jax.jit, static_argnames=())
def _ploss(outputs, labels, global_logit):
    n, k = outputs.shape
    blk = 512
    nblocks = n // blk
    labels2d = labels.reshape(n, 1)
    out = pl.pallas_call(
        functools.partial(_ploss_block, nblocks=nblocks, n_rows=n),
        grid=(nblocks,),
        in_specs=[
            pl.BlockSpec((blk, 1), lambda i: (i, 0)),
            pl.BlockSpec((blk, k), lambda i: (i, 0)),
            pl.BlockSpec(global_logit.shape, lambda i: (0, 0)),
        ],
        out_specs=pl.BlockSpec((1, 1), lambda i: (0, 0)),
        out_shape=jax.ShapeDtypeStruct((1, 1), jnp.float32),
    )(labels2d, outputs, global_logit)
    return out[0, 0]


def kernel(outputs, labels, global_logit):
    return _ploss(outputs.astype(jnp.float32), labels, global_logit)
